# Initial kernel scaffold; baseline (speedup 1.0000x reference)
import jax
import jax.numpy as jnp
from jax import lax
from jax.experimental import pallas as pl
from jax.experimental.pallas import tpu as pltpu
from jax.experimental.pallas import tpu_sc as plsc

N = 10000
E = 320000
D_IN = 128
D_HID = 128
D_OUT = 40
D_PAD = 64

NC, NS = 2, 16
NW = NC * NS
EPW = E // NW
K = 80
CHUNKS = EPW // K
ROW_STRIDE = 624
ROW_SPAN = 640

_mesh = plsc.VectorSubcoreMesh(core_axis_name="c", subcore_axis_name="s")


def _deg_body(dst_hbm, zero_hbm, out_a, out_b, ones_v, idx_v, acc):
    cid = lax.axis_index("c")
    sid = lax.axis_index("s")
    wid = sid * NC + cid
    base = wid * EPW
    off = sid * ROW_STRIDE
    for j in range(K // 16):
        ones_v[pl.ds(j * 16, 16)] = jnp.full((16,), 1.0, jnp.float32)
    pltpu.sync_copy(zero_hbm.at[pl.ds(off, ROW_SPAN)],
                    acc.at[pl.ds(off, ROW_SPAN)])
    plsc.subcore_barrier()

    def chunk(i, carry):
        eoff = pl.multiple_of(base + i * K, 8)
        pltpu.sync_copy(dst_hbm.at[pl.ds(eoff, K)], idx_v)
        pltpu.sync_copy(ones_v, acc.at[idx_v], add=True)
        return carry

    lax.fori_loop(0, CHUNKS, chunk, 0)
    plsc.subcore_barrier()

    @pl.when(cid == 0)
    def _():
        pltpu.sync_copy(acc.at[pl.ds(off, ROW_SPAN)],
                        out_a.at[pl.ds(off, ROW_SPAN)])

    @pl.when(cid == 1)
    def _():
        pltpu.sync_copy(acc.at[pl.ds(off, ROW_SPAN)],
                        out_b.at[pl.ds(off, ROW_SPAN)])


_deg_call = pl.kernel(
    _deg_body,
    out_type=[jax.ShapeDtypeStruct((N,), jnp.float32),
              jax.ShapeDtypeStruct((N,), jnp.float32)],
    mesh=_mesh,
    scratch_types=[
        pltpu.VMEM((K,), jnp.float32),
        pltpu.VMEM((K,), jnp.int32),
        pltpu.VMEM_SHARED((N,), jnp.float32),
    ],
)


def _agg_body(hp_hbm, src_hbm, dst_hbm, zero_hbm, out_a, out_b,
              idx_s, idx_d, rows, acc):
    cid = lax.axis_index("c")
    sid = lax.axis_index("s")
    wid = sid * NC + cid
    base = wid * EPW
    off = sid * ROW_STRIDE
    pltpu.sync_copy(zero_hbm.at[pl.ds(off, ROW_SPAN)],
                    acc.at[pl.ds(off, ROW_SPAN)])
    plsc.subcore_barrier()

    def chunk(i, carry):
        eoff = pl.multiple_of(base + i * K, 8)
        pltpu.sync_copy(src_hbm.at[pl.ds(eoff, K)], idx_s)
        pltpu.sync_copy(dst_hbm.at[pl.ds(eoff, K)], idx_d)
        pltpu.sync_copy(hp_hbm.at[idx_s], rows)
        pltpu.sync_copy(rows, acc.at[idx_d], add=True)
        return carry

    lax.fori_loop(0, CHUNKS, chunk, 0)
    plsc.subcore_barrier()

    @pl.when(cid == 0)
    def _():
        pltpu.sync_copy(acc.at[pl.ds(off, ROW_SPAN)],
                        out_a.at[pl.ds(off, ROW_SPAN)])

    @pl.when(cid == 1)
    def _():
        pltpu.sync_copy(acc.at[pl.ds(off, ROW_SPAN)],
                        out_b.at[pl.ds(off, ROW_SPAN)])


def _make_agg(d):
    return pl.kernel(
        _agg_body,
        out_type=[jax.ShapeDtypeStruct((N, d), jnp.float32),
                  jax.ShapeDtypeStruct((N, d), jnp.float32)],
        mesh=_mesh,
        scratch_types=[
            pltpu.VMEM((K,), jnp.int32),
            pltpu.VMEM((K,), jnp.int32),
            pltpu.VMEM((K, d), jnp.float32),
            pltpu.VMEM_SHARED((N, d), jnp.float32),
        ],
    )


_agg128 = _make_agg(D_HID)
_agg64 = _make_agg(D_PAD)


_R = 500
_G = N // _R


def _tc1_body(x_ref, w_ref, da_ref, db_ref, hp_ref, dinv_ref):
    d = da_ref[...] + db_ref[...] + 1.0
    dinv = lax.rsqrt(d)
    dinv_ref[...] = dinv
    h = jnp.dot(x_ref[...], w_ref[...], preferred_element_type=jnp.float32)
    hp_ref[...] = h * dinv


def _tc1(x, w, da, db):
    return pl.pallas_call(
        _tc1_body,
        grid=(_G,),
        in_specs=[
            pl.BlockSpec((_R, D_IN), lambda i: (i, 0)),
            pl.BlockSpec((D_IN, D_HID), lambda i: (0, 0)),
            pl.BlockSpec((_R, 1), lambda i: (i, 0)),
            pl.BlockSpec((_R, 1), lambda i: (i, 0)),
        ],
        out_specs=[
            pl.BlockSpec((_R, D_HID), lambda i: (i, 0)),
            pl.BlockSpec((_R, 1), lambda i: (i, 0)),
        ],
        out_shape=[jax.ShapeDtypeStruct((N, D_HID), jnp.float32),
                   jax.ShapeDtypeStruct((N, 1), jnp.float32)],
    )(x, w, da, db)


def _tc_mid_body(aa_ref, ab_ref, hp_ref, b_ref, dinv_ref, w_ref, out_ref):
    dinv = dinv_ref[...]
    h = dinv * (aa_ref[...] + ab_ref[...] + hp_ref[...]) + b_ref[...]
    h = jnp.maximum(h, 0.0)
    out_ref[...] = jnp.dot(h, w_ref[...],
                           preferred_element_type=jnp.float32) * dinv


def _tc_mid(aa, ab, hp, b, dinv, w, d_in, d_out):
    return pl.pallas_call(
        _tc_mid_body,
        grid=(_G,),
        in_specs=[
            pl.BlockSpec((_R, d_in), lambda i: (i, 0)),
            pl.BlockSpec((_R, d_in), lambda i: (i, 0)),
            pl.BlockSpec((_R, d_in), lambda i: (i, 0)),
            pl.BlockSpec((1, d_in), lambda i: (0, 0)),
            pl.BlockSpec((_R, 1), lambda i: (i, 0)),
            pl.BlockSpec((d_in, d_out), lambda i: (0, 0)),
        ],
        out_specs=pl.BlockSpec((_R, d_out), lambda i: (i, 0)),
        out_shape=jax.ShapeDtypeStruct((N, d_out), jnp.float32),
    )(aa, ab, hp, b, dinv, w)


def _tc_fin_body(aa_ref, ab_ref, hp_ref, b_ref, dinv_ref, out_ref):
    h = dinv_ref[...] * (aa_ref[...] + ab_ref[...] + hp_ref[...]) + b_ref[...]
    mask = lax.broadcasted_iota(jnp.int32, (_R, D_PAD), 1) < D_OUT
    hm = jnp.where(mask, h, -jnp.inf)
    m = jnp.max(hm, axis=1, keepdims=True)
    s = jnp.sum(jnp.where(mask, jnp.exp(h - m), 0.0), axis=1, keepdims=True)
    out_ref[...] = h - (jnp.log(s) + m)


def _tc_fin(aa, ab, hp, b, dinv):
    return pl.pallas_call(
        _tc_fin_body,
        grid=(_G,),
        in_specs=[
            pl.BlockSpec((_R, D_PAD), lambda i: (i, 0)),
            pl.BlockSpec((_R, D_PAD), lambda i: (i, 0)),
            pl.BlockSpec((_R, D_PAD), lambda i: (i, 0)),
            pl.BlockSpec((1, D_PAD), lambda i: (0, 0)),
            pl.BlockSpec((_R, 1), lambda i: (i, 0)),
        ],
        out_specs=pl.BlockSpec((_R, D_PAD), lambda i: (i, 0)),
        out_shape=jax.ShapeDtypeStruct((N, D_PAD), jnp.float32),
    )(aa, ab, hp, b, dinv)


def kernel(x, edge_index, W1, b1, W2, b2, W3, b3):
    src = edge_index[0]
    dst = edge_index[1]
    zero1 = jnp.zeros((N,), jnp.float32)
    zero128 = jnp.zeros((N, D_HID), jnp.float32)
    zero64 = jnp.zeros((N, D_PAD), jnp.float32)

    dega, degb = _deg_call(dst, zero1)
    hp1, dinv = _tc1(x, W1, dega.reshape(N, 1), degb.reshape(N, 1))

    a1a, a1b = _agg128(hp1, src, dst, zero128)
    hp2 = _tc_mid(a1a, a1b, hp1, b1.reshape(1, D_HID), dinv, W2,
                  D_HID, D_HID)

    a2a, a2b = _agg128(hp2, src, dst, zero128)
    W3p = jnp.pad(W3, ((0, 0), (0, D_PAD - D_OUT)))
    hp3 = _tc_mid(a2a, a2b, hp2, b2.reshape(1, D_HID), dinv, W3p,
                  D_HID, D_PAD)

    a3a, a3b = _agg64(hp3, src, dst, zero64)
    b3p = jnp.pad(b3, (0, D_PAD - D_OUT)).reshape(1, D_PAD)
    out = _tc_fin(a3a, a3b, hp3, b3p, dinv)
    return out[:, :D_OUT]


# speedup vs baseline: 9.3573x; 9.3573x over previous
import jax
import jax.numpy as jnp
from jax import lax
from jax.experimental import pallas as pl
from jax.experimental.pallas import tpu as pltpu
from jax.experimental.pallas import tpu_sc as plsc

N = 10000
E = 320000
D_IN = 128
D_HID = 128
D_OUT = 40
D_PAD = 64

NC, NS = 2, 16
NW = NC * NS
EPW = E // NW
K = 80
CHUNKS = EPW // K
ROW_STRIDE = 624
ROW_SPAN = 640

_mesh = plsc.VectorSubcoreMesh(core_axis_name="c", subcore_axis_name="s")


def _deg_body(dst_hbm, zero_hbm, out_a, out_b, ones_v, idx_v, zbuf, acc):
    cid = lax.axis_index("c")
    sid = lax.axis_index("s")
    wid = sid * NC + cid
    base = wid * EPW
    off = sid * ROW_STRIDE
    for j in range(K // 16):
        ones_v[pl.ds(j * 16, 16)] = jnp.full((16,), 1.0, jnp.float32)
    pltpu.sync_copy(zero_hbm, zbuf)
    for q in range(ROW_SPAN // K):
        pltpu.sync_copy(zbuf, acc.at[pl.ds(off + q * K, K)])
    plsc.subcore_barrier()

    def chunk(i, carry):
        eoff = pl.multiple_of(base + i * K, 8)
        pltpu.sync_copy(dst_hbm.at[pl.ds(eoff, K)], idx_v)
        pltpu.sync_copy(ones_v, acc.at[idx_v], add=True)
        return carry

    lax.fori_loop(0, CHUNKS, chunk, 0)
    plsc.subcore_barrier()
    out = [out_a, out_b]
    for c in range(NC):
        @pl.when(cid == c)
        def _(c=c):
            for q in range(ROW_SPAN // K):
                pltpu.sync_copy(acc.at[pl.ds(off + q * K, K)], zbuf)
                pltpu.sync_copy(zbuf, out[c].at[pl.ds(off + q * K, K)])


_deg_call = pl.kernel(
    _deg_body,
    out_type=[jax.ShapeDtypeStruct((N,), jnp.float32),
              jax.ShapeDtypeStruct((N,), jnp.float32)],
    mesh=_mesh,
    scratch_types=[
        pltpu.VMEM((K,), jnp.float32),
        pltpu.VMEM((K,), jnp.int32),
        pltpu.VMEM((K,), jnp.float32),
        pltpu.VMEM_SHARED((N,), jnp.float32),
    ],
)


def _agg_body(hp_hbm, src_hbm, dst_hbm, zero_hbm, out_a, out_b,
              idx_s, idx_d, rows, acc):
    cid = lax.axis_index("c")
    sid = lax.axis_index("s")
    wid = sid * NC + cid
    base = wid * EPW
    off = sid * ROW_STRIDE
    pltpu.sync_copy(zero_hbm, rows)
    for q in range(ROW_SPAN // K):
        pltpu.sync_copy(rows, acc.at[pl.ds(off + q * K, K)])
    plsc.subcore_barrier()

    def chunk(i, carry):
        eoff = pl.multiple_of(base + i * K, 8)
        pltpu.sync_copy(src_hbm.at[pl.ds(eoff, K)], idx_s)
        pltpu.sync_copy(dst_hbm.at[pl.ds(eoff, K)], idx_d)
        pltpu.sync_copy(hp_hbm.at[idx_s], rows)
        pltpu.sync_copy(rows, acc.at[idx_d], add=True)
        return carry

    lax.fori_loop(0, CHUNKS, chunk, 0)
    plsc.subcore_barrier()
    out = [out_a, out_b]
    for c in range(NC):
        @pl.when(cid == c)
        def _(c=c):
            for q in range(ROW_SPAN // K):
                pltpu.sync_copy(acc.at[pl.ds(off + q * K, K)], rows)
                pltpu.sync_copy(rows, out[c].at[pl.ds(off + q * K, K)])


def _make_agg(d):
    return pl.kernel(
        _agg_body,
        out_type=[jax.ShapeDtypeStruct((N, d), jnp.float32),
                  jax.ShapeDtypeStruct((N, d), jnp.float32)],
        mesh=_mesh,
        compiler_params=pltpu.CompilerParams(use_tc_tiling_on_sc=False),
        scratch_types=[
            pltpu.VMEM((K,), jnp.int32),
            pltpu.VMEM((K,), jnp.int32),
            pltpu.VMEM((K, d), jnp.float32),
            pltpu.VMEM_SHARED((N, d), jnp.float32),
        ],
    )


_agg128 = _make_agg(D_HID)
_agg64 = _make_agg(D_PAD)


_R = 1000
_G = N // _R


def _tc1_body(x_ref, w_ref, da_ref, db_ref, hp_ref, dinv_ref):
    d = da_ref[...] + db_ref[...] + 1.0
    dinv = lax.rsqrt(d)
    dinv_ref[...] = dinv
    h = jnp.dot(x_ref[...], w_ref[...], preferred_element_type=jnp.float32)
    hp_ref[...] = h * dinv


def _tc1(x, w, da, db):
    return pl.pallas_call(
        _tc1_body,
        grid=(_G,),
        in_specs=[
            pl.BlockSpec((_R, D_IN), lambda i: (i, 0)),
            pl.BlockSpec((D_IN, D_HID), lambda i: (0, 0)),
            pl.BlockSpec((_R, 1), lambda i: (i, 0)),
            pl.BlockSpec((_R, 1), lambda i: (i, 0)),
        ],
        out_specs=[
            pl.BlockSpec((_R, D_HID), lambda i: (i, 0)),
            pl.BlockSpec((_R, 1), lambda i: (i, 0)),
        ],
        out_shape=[jax.ShapeDtypeStruct((N, D_HID), jnp.float32),
                   jax.ShapeDtypeStruct((N, 1), jnp.float32)],
    )(x, w, da, db)


def _tc_mid_body(aa_ref, ab_ref, hp_ref, b_ref, dinv_ref, w_ref, out_ref):
    dinv = dinv_ref[...]
    h = dinv * (aa_ref[...] + ab_ref[...] + hp_ref[...]) + b_ref[...]
    h = jnp.maximum(h, 0.0)
    out_ref[...] = jnp.dot(h, w_ref[...],
                           preferred_element_type=jnp.float32) * dinv


def _tc_mid(aa, ab, hp, b, dinv, w, d_in, d_out):
    return pl.pallas_call(
        _tc_mid_body,
        grid=(_G,),
        in_specs=[
            pl.BlockSpec((_R, d_in), lambda i: (i, 0)),
            pl.BlockSpec((_R, d_in), lambda i: (i, 0)),
            pl.BlockSpec((_R, d_in), lambda i: (i, 0)),
            pl.BlockSpec((1, d_in), lambda i: (0, 0)),
            pl.BlockSpec((_R, 1), lambda i: (i, 0)),
            pl.BlockSpec((d_in, d_out), lambda i: (0, 0)),
        ],
        out_specs=pl.BlockSpec((_R, d_out), lambda i: (i, 0)),
        out_shape=jax.ShapeDtypeStruct((N, d_out), jnp.float32),
    )(aa, ab, hp, b, dinv, w)


def _tc_fin_body(aa_ref, ab_ref, hp_ref, b_ref, dinv_ref, out_ref):
    h = dinv_ref[...] * (aa_ref[...] + ab_ref[...] + hp_ref[...]) + b_ref[...]
    mask = lax.broadcasted_iota(jnp.int32, (_R, D_PAD), 1) < D_OUT
    hm = jnp.where(mask, h, -jnp.inf)
    m = jnp.max(hm, axis=1, keepdims=True)
    s = jnp.sum(jnp.where(mask, jnp.exp(h - m), 0.0), axis=1, keepdims=True)
    out_ref[...] = h - (jnp.log(s) + m)


def _tc_fin(aa, ab, hp, b, dinv):
    return pl.pallas_call(
        _tc_fin_body,
        grid=(_G,),
        in_specs=[
            pl.BlockSpec((_R, D_PAD), lambda i: (i, 0)),
            pl.BlockSpec((_R, D_PAD), lambda i: (i, 0)),
            pl.BlockSpec((_R, D_PAD), lambda i: (i, 0)),
            pl.BlockSpec((1, D_PAD), lambda i: (0, 0)),
            pl.BlockSpec((_R, 1), lambda i: (i, 0)),
        ],
        out_specs=pl.BlockSpec((_R, D_PAD), lambda i: (i, 0)),
        out_shape=jax.ShapeDtypeStruct((N, D_PAD), jnp.float32),
    )(aa, ab, hp, b, dinv)


def kernel(x, edge_index, W1, b1, W2, b2, W3, b3):
    src = edge_index[0]
    dst = edge_index[1]
    zero1 = jnp.zeros((K,), jnp.float32)
    zero128 = jnp.zeros((K, D_HID), jnp.float32)
    zero64 = jnp.zeros((K, D_PAD), jnp.float32)

    dega, degb = _deg_call(dst, zero1)
    hp1, dinv = _tc1(x, W1, dega.reshape(N, 1), degb.reshape(N, 1))

    a1a, a1b = _agg128(hp1, src, dst, zero128)
    hp2 = _tc_mid(a1a, a1b, hp1, b1.reshape(1, D_HID), dinv, W2,
                  D_HID, D_HID)

    a2a, a2b = _agg128(hp2, src, dst, zero128)
    W3p = jnp.pad(W3, ((0, 0), (0, D_PAD - D_OUT)))
    hp3 = _tc_mid(a2a, a2b, hp2, b2.reshape(1, D_HID), dinv, W3p,
                  D_HID, D_PAD)

    a3a, a3b = _agg64(hp3, src, dst, zero64)
    b3p = jnp.pad(b3, (0, D_PAD - D_OUT)).reshape(1, D_PAD)
    out = _tc_fin(a3a, a3b, hp3, b3p, dinv)
    return out[:, :D_OUT]


# speedup vs baseline: 23.1176x; 2.4705x over previous
import jax
import jax.numpy as jnp
from jax import lax
from jax.experimental import pallas as pl
from jax.experimental.pallas import tpu as pltpu
from jax.experimental.pallas import tpu_sc as plsc

N = 10000
E = 320000
D_IN = 128
D_HID = 128
D_OUT = 40
D_PAD = 64

NC, NS = 2, 16
NW = NC * NS
EPW = E // NW
K = 80
CHUNKS = EPW // K
ROW_STRIDE = 624
ROW_SPAN = 640

_mesh = plsc.VectorSubcoreMesh(core_axis_name="c", subcore_axis_name="s")


def _deg_body(dst_hbm, zero_hbm, out_a, out_b, ones_v, dst_v, zbuf, acc,
              sem_s):
    cid = lax.axis_index("c")
    sid = lax.axis_index("s")
    wid = sid * NC + cid
    off = sid * ROW_STRIDE
    for j in range(K // 16):
        ones_v[pl.ds(j * 16, 16)] = jnp.full((16,), 1.0, jnp.float32)
    pltpu.sync_copy(dst_hbm.at[wid], dst_v)
    pltpu.sync_copy(zero_hbm, zbuf)
    for q in range(ROW_SPAN // K):
        pltpu.sync_copy(zbuf, acc.at[pl.ds(off + q * K, K)])
    plsc.subcore_barrier()

    _NB = 5

    def scat(i, b):
        pltpu.async_copy(ones_v, acc.at[dst_v.at[i]], sem_s.at[b], add=True)

    for b in range(_NB):
        scat(b, b)

    def group(g, carry):
        for b in range(_NB):
            i = g * _NB + b
            pltpu.make_async_copy(ones_v, acc.at[dst_v.at[i]],
                                  sem_s.at[b]).wait()
            scat(i + _NB, b)
        return carry

    lax.fori_loop(0, CHUNKS // _NB - 1, group, 0)
    for b in range(_NB):
        i = (CHUNKS // _NB - 1) * _NB + b
        pltpu.make_async_copy(ones_v, acc.at[dst_v.at[i]],
                              sem_s.at[b]).wait()
    plsc.subcore_barrier()
    out = [out_a, out_b]
    for c in range(NC):
        @pl.when(cid == c)
        def _(c=c):
            for q in range(ROW_SPAN // K):
                pltpu.sync_copy(acc.at[pl.ds(off + q * K, K)], zbuf)
                pltpu.sync_copy(zbuf, out[c].at[pl.ds(off + q * K, K)])


_deg_call = pl.kernel(
    _deg_body,
    out_type=[jax.ShapeDtypeStruct((N,), jnp.float32),
              jax.ShapeDtypeStruct((N,), jnp.float32)],
    mesh=_mesh,
    compiler_params=pltpu.CompilerParams(use_tc_tiling_on_sc=False),
    scratch_types=[
        pltpu.VMEM((K,), jnp.float32),
        pltpu.VMEM((CHUNKS, K), jnp.int32),
        pltpu.VMEM((K,), jnp.float32),
        pltpu.VMEM_SHARED((N,), jnp.float32),
        pltpu.SemaphoreType.DMA((5,)),
    ],
)


KA = 40
CHUNKSA = EPW // KA
NBUF = 5
GROUPS = CHUNKSA // NBUF


def _agg_body(hp_hbm, src_hbm, dst_hbm, zero_hbm, out_a, out_b,
              src_v, dst_v, rows, acc, sem_g, sem_s):
    cid = lax.axis_index("c")
    sid = lax.axis_index("s")
    wid = sid * NC + cid
    off = sid * ROW_STRIDE

    pltpu.sync_copy(src_hbm.at[wid], src_v)
    pltpu.sync_copy(dst_hbm.at[wid], dst_v)

    pltpu.sync_copy(zero_hbm, rows.at[0])
    for q in range(ROW_SPAN // KA):
        pltpu.sync_copy(rows.at[0], acc.at[pl.ds(off + q * KA, KA)])
    plsc.subcore_barrier()

    def gather(i, b):
        return pltpu.async_copy(hp_hbm.at[src_v.at[i]], rows.at[b],
                                sem_g.at[b])

    def scatter(i, b):
        return pltpu.async_copy(rows.at[b], acc.at[dst_v.at[i]],
                                sem_s.at[b], add=True)

    for b in range(NBUF):
        gather(b, b)

    def group(g, carry):
        for b in range(NBUF):
            i = g * NBUF + b
            pltpu.make_async_copy(hp_hbm.at[src_v.at[i]], rows.at[b],
                                  sem_g.at[b]).wait()
            scatter(i, b)
        for b in range(NBUF):
            i = g * NBUF + b
            pltpu.make_async_copy(rows.at[b], acc.at[dst_v.at[i]],
                                  sem_s.at[b]).wait()
            gather(i + NBUF, b)
        return carry

    lax.fori_loop(0, GROUPS - 1, group, 0)
    for b in range(NBUF):
        i = (GROUPS - 1) * NBUF + b
        pltpu.make_async_copy(hp_hbm.at[src_v.at[i]], rows.at[b],
                              sem_g.at[b]).wait()
        scatter(i, b)
    for b in range(NBUF):
        i = (GROUPS - 1) * NBUF + b
        pltpu.make_async_copy(rows.at[b], acc.at[dst_v.at[i]],
                              sem_s.at[b]).wait()

    plsc.subcore_barrier()
    out = [out_a, out_b]
    for c in range(NC):
        @pl.when(cid == c)
        def _(c=c):
            for q in range(ROW_SPAN // KA):
                pltpu.sync_copy(acc.at[pl.ds(off + q * KA, KA)], rows.at[0])
                pltpu.sync_copy(rows.at[0],
                                out[c].at[pl.ds(off + q * KA, KA)])


def _make_agg(d):
    return pl.kernel(
        _agg_body,
        out_type=[jax.ShapeDtypeStruct((N, d), jnp.float32),
                  jax.ShapeDtypeStruct((N, d), jnp.float32)],
        mesh=_mesh,
        compiler_params=pltpu.CompilerParams(use_tc_tiling_on_sc=False),
        scratch_types=[
            pltpu.VMEM((CHUNKSA, KA), jnp.int32),
            pltpu.VMEM((CHUNKSA, KA), jnp.int32),
            pltpu.VMEM((NBUF, KA, d), jnp.float32),
            pltpu.VMEM_SHARED((N, d), jnp.float32),
            pltpu.SemaphoreType.DMA((NBUF,)),
            pltpu.SemaphoreType.DMA((NBUF,)),
        ],
    )


_agg128 = _make_agg(D_HID)
_agg64 = _make_agg(D_PAD)


_R = 1000
_G = N // _R


def _tc1_body(x_ref, w_ref, da_ref, db_ref, hp_ref, dinv_ref):
    d = da_ref[...] + db_ref[...] + 1.0
    dinv = lax.rsqrt(d)
    dinv_ref[...] = dinv
    h = jnp.dot(x_ref[...], w_ref[...], preferred_element_type=jnp.float32)
    hp_ref[...] = h * dinv


def _tc1(x, w, da, db):
    return pl.pallas_call(
        _tc1_body,
        grid=(_G,),
        in_specs=[
            pl.BlockSpec((_R, D_IN), lambda i: (i, 0)),
            pl.BlockSpec((D_IN, D_HID), lambda i: (0, 0)),
            pl.BlockSpec((_R, 1), lambda i: (i, 0)),
            pl.BlockSpec((_R, 1), lambda i: (i, 0)),
        ],
        out_specs=[
            pl.BlockSpec((_R, D_HID), lambda i: (i, 0)),
            pl.BlockSpec((_R, 1), lambda i: (i, 0)),
        ],
        out_shape=[jax.ShapeDtypeStruct((N, D_HID), jnp.float32),
                   jax.ShapeDtypeStruct((N, 1), jnp.float32)],
    )(x, w, da, db)


def _tc_mid_body(aa_ref, ab_ref, hp_ref, b_ref, dinv_ref, w_ref, out_ref):
    dinv = dinv_ref[...]
    h = dinv * (aa_ref[...] + ab_ref[...] + hp_ref[...]) + b_ref[...]
    h = jnp.maximum(h, 0.0)
    out_ref[...] = jnp.dot(h, w_ref[...],
                           preferred_element_type=jnp.float32) * dinv


def _tc_mid(aa, ab, hp, b, dinv, w, d_in, d_out):
    return pl.pallas_call(
        _tc_mid_body,
        grid=(_G,),
        in_specs=[
            pl.BlockSpec((_R, d_in), lambda i: (i, 0)),
            pl.BlockSpec((_R, d_in), lambda i: (i, 0)),
            pl.BlockSpec((_R, d_in), lambda i: (i, 0)),
            pl.BlockSpec((1, d_in), lambda i: (0, 0)),
            pl.BlockSpec((_R, 1), lambda i: (i, 0)),
            pl.BlockSpec((d_in, d_out), lambda i: (0, 0)),
        ],
        out_specs=pl.BlockSpec((_R, d_out), lambda i: (i, 0)),
        out_shape=jax.ShapeDtypeStruct((N, d_out), jnp.float32),
    )(aa, ab, hp, b, dinv, w)


def _tc_fin_body(aa_ref, ab_ref, hp_ref, b_ref, dinv_ref, out_ref):
    h = dinv_ref[...] * (aa_ref[...] + ab_ref[...] + hp_ref[...]) + b_ref[...]
    mask = lax.broadcasted_iota(jnp.int32, (_R, D_PAD), 1) < D_OUT
    hm = jnp.where(mask, h, -jnp.inf)
    m = jnp.max(hm, axis=1, keepdims=True)
    s = jnp.sum(jnp.where(mask, jnp.exp(h - m), 0.0), axis=1, keepdims=True)
    out_ref[...] = h - (jnp.log(s) + m)


def _tc_fin(aa, ab, hp, b, dinv):
    return pl.pallas_call(
        _tc_fin_body,
        grid=(_G,),
        in_specs=[
            pl.BlockSpec((_R, D_PAD), lambda i: (i, 0)),
            pl.BlockSpec((_R, D_PAD), lambda i: (i, 0)),
            pl.BlockSpec((_R, D_PAD), lambda i: (i, 0)),
            pl.BlockSpec((1, D_PAD), lambda i: (0, 0)),
            pl.BlockSpec((_R, 1), lambda i: (i, 0)),
        ],
        out_specs=pl.BlockSpec((_R, D_PAD), lambda i: (i, 0)),
        out_shape=jax.ShapeDtypeStruct((N, D_PAD), jnp.float32),
    )(aa, ab, hp, b, dinv)


def kernel(x, edge_index, W1, b1, W2, b2, W3, b3):
    src = edge_index[0].reshape(NW, CHUNKSA, KA)
    dst = edge_index[1].reshape(NW, CHUNKSA, KA)
    dst80 = edge_index[1].reshape(NW, CHUNKS, K)
    zero1 = jnp.zeros((K,), jnp.float32)
    zero128 = jnp.zeros((KA, D_HID), jnp.float32)
    zero64 = jnp.zeros((KA, D_PAD), jnp.float32)

    dega, degb = _deg_call(dst80, zero1)
    hp1, dinv = _tc1(x, W1, dega.reshape(N, 1), degb.reshape(N, 1))

    a1a, a1b = _agg128(hp1, src, dst, zero128)
    hp2 = _tc_mid(a1a, a1b, hp1, b1.reshape(1, D_HID), dinv, W2,
                  D_HID, D_HID)

    a2a, a2b = _agg128(hp2, src, dst, zero128)
    W3p = jnp.pad(W3, ((0, 0), (0, D_PAD - D_OUT)))
    hp3 = _tc_mid(a2a, a2b, hp2, b2.reshape(1, D_HID), dinv, W3p,
                  D_HID, D_PAD)

    a3a, a3b = _agg64(hp3, src, dst, zero64)
    b3p = jnp.pad(b3, (0, D_PAD - D_OUT)).reshape(1, D_PAD)
    out = _tc_fin(a3a, a3b, hp3, b3p, dinv)
    return out[:, :D_OUT]


# speedup vs baseline: 23.8463x; 1.0315x over previous
import jax
import jax.numpy as jnp
from jax import lax
from jax.experimental import pallas as pl
from jax.experimental.pallas import tpu as pltpu
from jax.experimental.pallas import tpu_sc as plsc

N = 10000
E = 320000
D_IN = 128
D_HID = 128
D_OUT = 40
D_PAD = 64

NC, NS = 2, 16
NW = NC * NS
EPW = E // NW
K = 80
CHUNKS = EPW // K
KA = 40
CHUNKSA = EPW // KA
NBUF = 5
GROUPS = CHUNKSA // NBUF
ROW_STRIDE = 624
ROW_SPAN = 640

_mesh = plsc.VectorSubcoreMesh(core_axis_name="c", subcore_axis_name="s")


def _deg_body(dst_hbm, zero_hbm, out_a, out_b, ones_v, dst_v, zbuf, acc,
              sem_s):
    cid = lax.axis_index("c")
    sid = lax.axis_index("s")
    wid = sid * NC + cid
    off = sid * ROW_STRIDE
    for j in range(K // 16):
        ones_v[pl.ds(j * 16, 16)] = jnp.full((16,), 1.0, jnp.float32)
    pltpu.sync_copy(dst_hbm.at[wid], dst_v)
    pltpu.sync_copy(zero_hbm, zbuf)
    for q in range(ROW_SPAN // K):
        pltpu.sync_copy(zbuf, acc.at[pl.ds(off + q * K, K)])
    plsc.subcore_barrier()

    _NB = 5
    ones = ones_v.at[pl.ds(0, KA)]

    def scat(i, b):
        pltpu.async_copy(ones, acc.at[dst_v.at[i]], sem_s.at[b], add=True)

    for b in range(_NB):
        scat(b, b)

    def group(g, carry):
        for b in range(_NB):
            i = g * _NB + b
            pltpu.make_async_copy(ones, acc.at[dst_v.at[i]],
                                  sem_s.at[b]).wait()
            scat(i + _NB, b)
        return carry

    lax.fori_loop(0, CHUNKSA // _NB - 1, group, 0)
    for b in range(_NB):
        i = (CHUNKSA // _NB - 1) * _NB + b
        pltpu.make_async_copy(ones, acc.at[dst_v.at[i]],
                              sem_s.at[b]).wait()
    plsc.subcore_barrier()
    out = [out_a, out_b]
    for c in range(NC):
        @pl.when(cid == c)
        def _(c=c):
            for q in range(ROW_SPAN // K):
                pltpu.sync_copy(acc.at[pl.ds(off + q * K, K)], zbuf)
                pltpu.sync_copy(zbuf, out[c].at[pl.ds(off + q * K, K)])


_deg_call = pl.kernel(
    _deg_body,
    out_type=[jax.ShapeDtypeStruct((N,), jnp.float32),
              jax.ShapeDtypeStruct((N,), jnp.float32)],
    mesh=_mesh,
    compiler_params=pltpu.CompilerParams(use_tc_tiling_on_sc=False),
    scratch_types=[
        pltpu.VMEM((K,), jnp.float32),
        pltpu.VMEM((CHUNKSA, KA), jnp.int32),
        pltpu.VMEM((K,), jnp.float32),
        pltpu.VMEM_SHARED((N,), jnp.float32),
        pltpu.SemaphoreType.DMA((5,)),
    ],
)


def _agg_body(hp_hbm, src_hbm, dst_hbm, zero_hbm, out_a, out_b,
              src_v, dst_v, rows, acc, sem_g, sem_s):
    cid = lax.axis_index("c")
    sid = lax.axis_index("s")
    wid = sid * NC + cid
    off = sid * ROW_STRIDE

    pltpu.async_copy(src_hbm.at[wid], src_v, sem_g.at[0])
    pltpu.async_copy(dst_hbm.at[wid], dst_v, sem_g.at[1])

    pltpu.sync_copy(zero_hbm, rows.at[0])
    for q in range(ROW_SPAN // KA):
        pltpu.sync_copy(rows.at[0], acc.at[pl.ds(off + q * KA, KA)])
    pltpu.make_async_copy(src_hbm.at[wid], src_v, sem_g.at[0]).wait()
    pltpu.make_async_copy(dst_hbm.at[wid], dst_v, sem_g.at[1]).wait()

    def gather(i, b):
        return pltpu.async_copy(hp_hbm.at[src_v.at[i]], rows.at[b],
                                sem_g.at[b])

    def scatter(i, b):
        return pltpu.async_copy(rows.at[b], acc.at[dst_v.at[i]],
                                sem_s.at[b], add=True)

    for b in range(NBUF):
        gather(b, b)
    plsc.subcore_barrier()

    def group(g, carry):
        for b in range(NBUF):
            i = g * NBUF + b
            pltpu.make_async_copy(hp_hbm.at[src_v.at[i]], rows.at[b],
                                  sem_g.at[b]).wait()
            scatter(i, b)
        for b in range(NBUF):
            i = g * NBUF + b
            pltpu.make_async_copy(rows.at[b], acc.at[dst_v.at[i]],
                                  sem_s.at[b]).wait()
            gather(i + NBUF, b)
        return carry

    lax.fori_loop(0, GROUPS - 1, group, 0)
    for b in range(NBUF):
        i = (GROUPS - 1) * NBUF + b
        pltpu.make_async_copy(hp_hbm.at[src_v.at[i]], rows.at[b],
                              sem_g.at[b]).wait()
        scatter(i, b)
    for b in range(NBUF):
        i = (GROUPS - 1) * NBUF + b
        pltpu.make_async_copy(rows.at[b], acc.at[dst_v.at[i]],
                              sem_s.at[b]).wait()

    plsc.subcore_barrier()
    out = [out_a, out_b]
    nq = ROW_SPAN // KA
    for c in range(NC):
        @pl.when(cid == c)
        def _(c=c):
            def rd(q, s):
                pltpu.async_copy(acc.at[pl.ds(off + q * KA, KA)],
                                 rows.at[s], sem_g.at[s])

            def rd_wait(q, s):
                pltpu.make_async_copy(acc.at[pl.ds(off + q * KA, KA)],
                                      rows.at[s], sem_g.at[s]).wait()

            def wr(q, s):
                pltpu.async_copy(rows.at[s],
                                 out[c].at[pl.ds(off + q * KA, KA)],
                                 sem_s.at[s])

            def wr_wait(q, s):
                pltpu.make_async_copy(rows.at[s],
                                      out[c].at[pl.ds(off + q * KA, KA)],
                                      sem_s.at[s]).wait()

            rd(0, 0)
            rd(1, 1)
            for q in range(nq):
                s = q % 2
                rd_wait(q, s)
                wr(q, s)
                if q + 2 < nq:
                    wr_wait(q, s)
                    rd(q + 2, s)
            wr_wait(nq - 2, (nq - 2) % 2)
            wr_wait(nq - 1, (nq - 1) % 2)


def _make_agg(d):
    return pl.kernel(
        _agg_body,
        out_type=[jax.ShapeDtypeStruct((N, d), jnp.float32),
                  jax.ShapeDtypeStruct((N, d), jnp.float32)],
        mesh=_mesh,
        compiler_params=pltpu.CompilerParams(use_tc_tiling_on_sc=False),
        scratch_types=[
            pltpu.VMEM((CHUNKSA, KA), jnp.int32),
            pltpu.VMEM((CHUNKSA, KA), jnp.int32),
            pltpu.VMEM((NBUF, KA, d), jnp.float32),
            pltpu.VMEM_SHARED((N, d), jnp.float32),
            pltpu.SemaphoreType.DMA((NBUF,)),
            pltpu.SemaphoreType.DMA((NBUF,)),
        ],
    )


_agg128 = _make_agg(D_HID)
_agg64 = _make_agg(D_PAD)


_R = 1000
_G = N // _R


def _tc1_body(x_ref, w_ref, da_ref, db_ref, hp_ref, dinv_ref):
    d = da_ref[...] + db_ref[...] + 1.0
    dinv = lax.rsqrt(d)
    dinv_ref[...] = dinv
    h = jnp.dot(x_ref[...], w_ref[...], preferred_element_type=jnp.float32)
    hp_ref[...] = h * dinv


def _tc1(x, w, da, db):
    return pl.pallas_call(
        _tc1_body,
        grid=(_G,),
        in_specs=[
            pl.BlockSpec((_R, D_IN), lambda i: (i, 0)),
            pl.BlockSpec((D_IN, D_HID), lambda i: (0, 0)),
            pl.BlockSpec((_R, 1), lambda i: (i, 0)),
            pl.BlockSpec((_R, 1), lambda i: (i, 0)),
        ],
        out_specs=[
            pl.BlockSpec((_R, D_HID), lambda i: (i, 0)),
            pl.BlockSpec((_R, 1), lambda i: (i, 0)),
        ],
        out_shape=[jax.ShapeDtypeStruct((N, D_HID), jnp.float32),
                   jax.ShapeDtypeStruct((N, 1), jnp.float32)],
    )(x, w, da, db)


def _tc_mid_body(aa_ref, ab_ref, hp_ref, b_ref, dinv_ref, w_ref, out_ref):
    dinv = dinv_ref[...]
    h = dinv * (aa_ref[...] + ab_ref[...] + hp_ref[...]) + b_ref[...]
    h = jnp.maximum(h, 0.0)
    out_ref[...] = jnp.dot(h, w_ref[...],
                           preferred_element_type=jnp.float32) * dinv


def _tc_mid(aa, ab, hp, b, dinv, w, d_in, d_out):
    return pl.pallas_call(
        _tc_mid_body,
        grid=(_G,),
        in_specs=[
            pl.BlockSpec((_R, d_in), lambda i: (i, 0)),
            pl.BlockSpec((_R, d_in), lambda i: (i, 0)),
            pl.BlockSpec((_R, d_in), lambda i: (i, 0)),
            pl.BlockSpec((1, d_in), lambda i: (0, 0)),
            pl.BlockSpec((_R, 1), lambda i: (i, 0)),
            pl.BlockSpec((d_in, d_out), lambda i: (0, 0)),
        ],
        out_specs=pl.BlockSpec((_R, d_out), lambda i: (i, 0)),
        out_shape=jax.ShapeDtypeStruct((N, d_out), jnp.float32),
    )(aa, ab, hp, b, dinv, w)


def _tc_fin_body(aa_ref, ab_ref, hp_ref, b_ref, dinv_ref, out_ref):
    h = dinv_ref[...] * (aa_ref[...] + ab_ref[...] + hp_ref[...]) + b_ref[...]
    mask = lax.broadcasted_iota(jnp.int32, (_R, D_PAD), 1) < D_OUT
    hm = jnp.where(mask, h, -jnp.inf)
    m = jnp.max(hm, axis=1, keepdims=True)
    s = jnp.sum(jnp.where(mask, jnp.exp(h - m), 0.0), axis=1, keepdims=True)
    out_ref[...] = h - (jnp.log(s) + m)


def _tc_fin(aa, ab, hp, b, dinv):
    return pl.pallas_call(
        _tc_fin_body,
        grid=(_G,),
        in_specs=[
            pl.BlockSpec((_R, D_PAD), lambda i: (i, 0)),
            pl.BlockSpec((_R, D_PAD), lambda i: (i, 0)),
            pl.BlockSpec((_R, D_PAD), lambda i: (i, 0)),
            pl.BlockSpec((1, D_PAD), lambda i: (0, 0)),
            pl.BlockSpec((_R, 1), lambda i: (i, 0)),
        ],
        out_specs=pl.BlockSpec((_R, D_PAD), lambda i: (i, 0)),
        out_shape=jax.ShapeDtypeStruct((N, D_PAD), jnp.float32),
    )(aa, ab, hp, b, dinv)


def kernel(x, edge_index, W1, b1, W2, b2, W3, b3):
    src = edge_index[0].reshape(NW, CHUNKSA, KA)
    dst = edge_index[1].reshape(NW, CHUNKSA, KA)
    zero1 = jnp.zeros((K,), jnp.float32)
    zero128 = jnp.zeros((KA, D_HID), jnp.float32)
    zero64 = jnp.zeros((KA, D_PAD), jnp.float32)

    dega, degb = _deg_call(dst, zero1)
    hp1, dinv = _tc1(x, W1, dega.reshape(N, 1), degb.reshape(N, 1))

    a1a, a1b = _agg128(hp1, src, dst, zero128)
    hp2 = _tc_mid(a1a, a1b, hp1, b1.reshape(1, D_HID), dinv, W2,
                  D_HID, D_HID)

    a2a, a2b = _agg128(hp2, src, dst, zero128)
    W3p = jnp.pad(W3, ((0, 0), (0, D_PAD - D_OUT)))
    hp3 = _tc_mid(a2a, a2b, hp2, b2.reshape(1, D_HID), dinv, W3p,
                  D_HID, D_PAD)

    a3a, a3b = _agg64(hp3, src, dst, zero64)
    b3p = jnp.pad(b3, (0, D_PAD - D_OUT)).reshape(1, D_PAD)
    out = _tc_fin(a3a, a3b, hp3, b3p, dinv)
    return out[:, :D_OUT]
